# baseline (speedup 1.0000x reference)
import functools

import jax
import jax.numpy as jnp
from jax import lax
from jax.experimental import pallas as pl
from jax.experimental.pallas import tpu as pltpu
from jax.experimental.pallas import tpu_sc as plsc

N_NODES = 10000
D = 128
N_GRAPHS = 64
N_EDGES = 320000
BN_EPS = 1e-5

NC = 2
NS = 16
NW = NC * NS
E_PER_TILE = N_EDGES // NW
K = 40
KT = 80
DK = 80
ROWS_A = 640
ROWS_B = 400

_mesh = plsc.VectorSubcoreMesh(core_axis_name="c", subcore_axis_name="s")


R = 8
DR = 5
NOUT = 31
DEG_NOUT = E_PER_TILE // (DR * DK)


@functools.partial(
    pl.kernel,
    out_type=jax.ShapeDtypeStruct((NC * N_NODES,), jnp.float32),
    mesh=_mesh,
    scratch_types=[
        pltpu.VMEM_SHARED((N_NODES,), jnp.float32),
        pltpu.VMEM((DK,), jnp.float32),
        [pltpu.VMEM((DK,), jnp.int32) for _ in range(DR)],
        [pltpu.SemaphoreType.DMA for _ in range(DR)],
        [pltpu.SemaphoreType.DMA for _ in range(DR)],
        pltpu.VMEM((N_NODES,), jnp.float32),
    ],
)
def _deg_kernel(dst_hbm, out_hbm, deg_sh, ones_v, didx, isem,
                ssem, stage_v):
    cid = lax.axis_index("c")
    sid = lax.axis_index("s")
    wid = cid * NS + sid

    ones16 = jnp.ones((16,), jnp.float32)
    for j in range(DK // 16):
        ones_v[pl.ds(j * 16, 16)] = ones16

    zeros16 = jnp.zeros((16,), jnp.float32)

    def _zb(i, _):
        stage_v[pl.ds(i * 16, 16)] = zeros16
        return 0

    lax.fori_loop(0, N_NODES // 16, _zb, 0)

    @pl.when(sid == 0)
    def _():
        pltpu.sync_copy(stage_v, deg_sh)

    base = wid * E_PER_TILE

    def _start_idx(o, b):
        c0 = base + (o * DR + b) * DK
        pltpu.async_copy(dst_hbm.at[pl.ds(c0, DK)], didx[b], isem[b])

    for b in range(DR):
        _start_idx(0, b)

    plsc.subcore_barrier()

    def _outer(o, _):
        for b in range(DR):
            pltpu.make_async_copy(dst_hbm.at[pl.ds(base, DK)], didx[b],
                                  isem[b]).wait()
            pltpu.async_copy(ones_v, deg_sh.at[didx[b]], ssem[b], add=True)
        for b in range(DR):
            pltpu.make_async_copy(ones_v, deg_sh.at[didx[b]], ssem[b]).wait()

            @pl.when(o < DEG_NOUT - 1)
            def _():
                _start_idx(o + 1, b)

        return 0

    lax.fori_loop(0, DEG_NOUT, _outer, 0)
    plsc.subcore_barrier()

    @pl.when(sid == 0)
    def _():
        pltpu.sync_copy(deg_sh, stage_v)
        pltpu.sync_copy(stage_v, out_hbm.at[pl.ds(cid * N_NODES, N_NODES)])


def _hs_body(x_ref, w_ref, degp_ref, hs_ref):
    h = jnp.dot(x_ref[...], w_ref[...], preferred_element_type=jnp.float32)
    deg = 1.0 + jnp.sum(degp_ref[...], axis=1)
    dinv = lax.rsqrt(deg)
    hs_ref[...] = h * dinv[:, None]


NB = 10
BN_ROWS = N_NODES // NB


def _hs_call(x, W, deg_parts):
    return pl.pallas_call(
        _hs_body,
        grid=(NB,),
        in_specs=[
            pl.BlockSpec((BN_ROWS, D), lambda i: (i, 0)),
            pl.BlockSpec((D, D), lambda i: (0, 0)),
            pl.BlockSpec((BN_ROWS, NC), lambda i: (i, 0)),
        ],
        out_specs=pl.BlockSpec((BN_ROWS, D), lambda i: (i, 0)),
        out_shape=jax.ShapeDtypeStruct((N_NODES, D), jnp.float32),
    )(x, W, deg_parts)


@functools.partial(
    pl.kernel,
    out_type=jax.ShapeDtypeStruct((NC, N_NODES, D), jnp.float32),
    mesh=_mesh,
    scratch_types=[
        pltpu.VMEM_SHARED((N_NODES, D), jnp.float32),
        [pltpu.VMEM((K,), jnp.int32) for _ in range(R)],
        [pltpu.VMEM((K,), jnp.int32) for _ in range(R)],
        [pltpu.VMEM((K, D), jnp.float32) for _ in range(R)],
        pltpu.VMEM((KT,), jnp.int32),
        pltpu.VMEM((KT,), jnp.int32),
        [pltpu.SemaphoreType.DMA for _ in range(R)],
        [pltpu.SemaphoreType.DMA for _ in range(R)],
        [pltpu.SemaphoreType.DMA for _ in range(R)],
    ],
)
def _acc_kernel(src_hbm, dst_hbm, hs_hbm, out_hbm,
                acc_sh, sidx, didx, rows, tsidx, tdidx, isem, gsem, ssem):
    cid = lax.axis_index("c")
    sid = lax.axis_index("s")
    wid = cid * NS + sid
    row0 = sid * ROWS_A
    base = wid * E_PER_TILE

    def _start_idx(o, b):
        c0 = base + (o * R + b) * K
        pltpu.async_copy(src_hbm.at[pl.ds(c0, K)], sidx[b], isem[b])
        pltpu.async_copy(dst_hbm.at[pl.ds(c0, K)], didx[b], isem[b])

    for b in range(R):
        _start_idx(0, b)

    zeros16 = jnp.zeros((16,), jnp.float32)
    for r in range(K):
        for c in range(D // 16):
            rows[0][r, pl.ds(c * 16, 16)] = zeros16

    def _zero_rows(i, _):
        pltpu.sync_copy(rows[0], acc_sh.at[pl.ds(row0 + i * K, K)])
        return 0

    @pl.when(sid < NS - 1)
    def _():
        lax.fori_loop(0, ROWS_A // K, _zero_rows, 0)

    @pl.when(sid == NS - 1)
    def _():
        lax.fori_loop(0, ROWS_B // K, _zero_rows, 0)

    plsc.subcore_barrier()

    def _outer(o, _):
        for b in range(R):
            pltpu.make_async_copy(src_hbm.at[pl.ds(base, K)], sidx[b],
                                  isem[b]).wait()
            pltpu.make_async_copy(dst_hbm.at[pl.ds(base, K)], didx[b],
                                  isem[b]).wait()
            pltpu.async_copy(hs_hbm.at[sidx[b]], rows[b], gsem[b])
        for b in range(R):
            pltpu.make_async_copy(hs_hbm.at[sidx[b]], rows[b], gsem[b]).wait()
            pltpu.async_copy(rows[b], acc_sh.at[didx[b]], ssem[b], add=True)
        for b in range(R):
            pltpu.make_async_copy(rows[b], acc_sh.at[didx[b]], ssem[b]).wait()

            @pl.when(o < NOUT - 1)
            def _():
                _start_idx(o + 1, b)

        return 0

    lax.fori_loop(0, NOUT, _outer, 0)

    t0 = base + NOUT * R * K
    pltpu.sync_copy(src_hbm.at[pl.ds(t0, KT)], tsidx)
    pltpu.sync_copy(dst_hbm.at[pl.ds(t0, KT)], tdidx)
    pltpu.async_copy(hs_hbm.at[tsidx], rows[0].at[pl.ds(0, KT)], gsem[0]).wait()
    pltpu.sync_copy(rows[0].at[pl.ds(0, KT)], acc_sh.at[tdidx], add=True)

    plsc.subcore_barrier()

    @pl.when(sid < NS - 1)
    def _():
        pltpu.sync_copy(acc_sh.at[pl.ds(row0, ROWS_A)],
                        out_hbm.at[cid, pl.ds(row0, ROWS_A)])

    @pl.when(sid == NS - 1)
    def _():
        pltpu.sync_copy(acc_sh.at[pl.ds(row0, ROWS_B)],
                        out_hbm.at[cid, pl.ds(row0, ROWS_B)])


def _final_body(accp_ref, x_ref, w_ref, degp_ref, batch_ref, b_ref, gamma_ref,
                beta_ref, mean_ref, var_ref, out_ref, sums_acc, counts_acc):
    deg = 1.0 + jnp.sum(degp_ref[...], axis=1)
    dinv = lax.rsqrt(deg)
    h = jnp.dot(x_ref[...], w_ref[...], preferred_element_type=jnp.float32)
    hs = h * dinv[:, None]
    acc = accp_ref[0] + accp_ref[1] + hs
    conv = acc * dinv[:, None] + b_ref[...]
    scale = gamma_ref[...] * lax.rsqrt(var_ref[...] + BN_EPS)
    shift = beta_ref[...] - mean_ref[...] * scale
    hbn = jnp.maximum(conv * scale + shift, 0.0)
    bt = batch_ref[...].reshape(BN_ROWS)
    onehot = (bt[None, :] == lax.broadcasted_iota(
        jnp.int32, (N_GRAPHS, BN_ROWS), 0)).astype(jnp.float32)
    sums = jnp.dot(onehot, hbn, preferred_element_type=jnp.float32)
    counts = jnp.sum(onehot, axis=1)

    i = pl.program_id(0)

    @pl.when(i == 0)
    def _():
        sums_acc[...] = jnp.zeros((N_GRAPHS, D), jnp.float32)
        counts_acc[...] = jnp.zeros((N_GRAPHS, D), jnp.float32)

    sums_acc[...] += sums
    counts_acc[...] += counts[:, None]

    @pl.when(i == NB - 1)
    def _():
        out_ref[...] = sums_acc[...] / jnp.clip(counts_acc[...], 1.0, None)


def _final_call(acc_parts, x, W, deg_parts, batch32, b, gamma, beta, mean, var):
    return pl.pallas_call(
        _final_body,
        grid=(NB,),
        in_specs=[
            pl.BlockSpec((NC, BN_ROWS, D), lambda i: (0, i, 0)),
            pl.BlockSpec((BN_ROWS, D), lambda i: (i, 0)),
            pl.BlockSpec((D, D), lambda i: (0, 0)),
            pl.BlockSpec((BN_ROWS, NC), lambda i: (i, 0)),
            pl.BlockSpec((1, 1, BN_ROWS), lambda i: (i, 0, 0)),
            pl.BlockSpec((1, D), lambda i: (0, 0)),
            pl.BlockSpec((1, D), lambda i: (0, 0)),
            pl.BlockSpec((1, D), lambda i: (0, 0)),
            pl.BlockSpec((1, D), lambda i: (0, 0)),
            pl.BlockSpec((1, D), lambda i: (0, 0)),
        ],
        out_specs=pl.BlockSpec((N_GRAPHS, D), lambda i: (0, 0)),
        out_shape=jax.ShapeDtypeStruct((N_GRAPHS, D), jnp.float32),
        scratch_shapes=[
            pltpu.VMEM((N_GRAPHS, D), jnp.float32),
            pltpu.VMEM((N_GRAPHS, D), jnp.float32),
        ],
    )(acc_parts, x, W, deg_parts, batch32.reshape(NB, 1, BN_ROWS),
      b.reshape(1, D), gamma.reshape(1, D), beta.reshape(1, D),
      mean.reshape(1, D), var.reshape(1, D))


def kernel(x, edge_index, batch, W, b, gamma, beta, running_mean, running_var):
    ei = edge_index.astype(jnp.int32)
    src = ei[0]
    dst = ei[1]
    batch32 = batch.astype(jnp.int32)

    deg_parts = _deg_kernel(dst).reshape(NC, N_NODES).T
    hs = _hs_call(x, W, deg_parts)
    acc_parts = _acc_kernel(src, dst, hs)
    return _final_call(acc_parts, x, W, deg_parts, batch32, b, gamma, beta,
                       running_mean, running_var)


# speedup vs baseline: 1.0051x; 1.0051x over previous
import functools

import jax
import jax.numpy as jnp
from jax import lax
from jax.experimental import pallas as pl
from jax.experimental.pallas import tpu as pltpu
from jax.experimental.pallas import tpu_sc as plsc

N_NODES = 10000
D = 128
N_GRAPHS = 64
N_EDGES = 320000
BN_EPS = 1e-5

NC = 2
NS = 16
NW = NC * NS
E_PER_TILE = N_EDGES // NW
K = 40
KT = 80
DK = 80
ROWS_A = 640
ROWS_B = 400

_mesh = plsc.VectorSubcoreMesh(core_axis_name="c", subcore_axis_name="s")


R = 8
DR = 5
NOUT = 31
DEG_NOUT = E_PER_TILE // (DR * DK)


@functools.partial(
    pl.kernel,
    out_type=jax.ShapeDtypeStruct((NC * N_NODES,), jnp.float32),
    mesh=_mesh,
    scratch_types=[
        pltpu.VMEM_SHARED((N_NODES,), jnp.float32),
        pltpu.VMEM((DK,), jnp.float32),
        [pltpu.VMEM((DK,), jnp.int32) for _ in range(DR)],
        [pltpu.SemaphoreType.DMA for _ in range(DR)],
        [pltpu.SemaphoreType.DMA for _ in range(DR)],
        pltpu.VMEM((N_NODES,), jnp.float32),
    ],
)
def _deg_kernel(dst_hbm, out_hbm, deg_sh, ones_v, didx, isem,
                ssem, stage_v):
    cid = lax.axis_index("c")
    sid = lax.axis_index("s")
    wid = cid * NS + sid

    ones16 = jnp.ones((16,), jnp.float32)
    for j in range(DK // 16):
        ones_v[pl.ds(j * 16, 16)] = ones16

    zeros16 = jnp.zeros((16,), jnp.float32)

    def _zb(i, _):
        stage_v[pl.ds(i * 16, 16)] = zeros16
        return 0

    lax.fori_loop(0, N_NODES // 16, _zb, 0)

    @pl.when(sid == 0)
    def _():
        pltpu.sync_copy(stage_v, deg_sh)

    base = wid * E_PER_TILE

    def _start_idx(o, b):
        c0 = base + (o * DR + b) * DK
        pltpu.async_copy(dst_hbm.at[pl.ds(c0, DK)], didx[b], isem[b])

    for b in range(DR):
        _start_idx(0, b)

    plsc.subcore_barrier()

    def _outer(o, _):
        for b in range(DR):
            pltpu.make_async_copy(dst_hbm.at[pl.ds(base, DK)], didx[b],
                                  isem[b]).wait()
            pltpu.async_copy(ones_v, deg_sh.at[didx[b]], ssem[b], add=True)
        for b in range(DR):
            pltpu.make_async_copy(ones_v, deg_sh.at[didx[b]], ssem[b]).wait()

            @pl.when(o < DEG_NOUT - 1)
            def _():
                _start_idx(o + 1, b)

        return 0

    lax.fori_loop(0, DEG_NOUT, _outer, 0)
    plsc.subcore_barrier()

    @pl.when(sid == 0)
    def _():
        pltpu.sync_copy(deg_sh, stage_v)
        pltpu.sync_copy(stage_v, out_hbm.at[pl.ds(cid * N_NODES, N_NODES)])


def _hs_body(x_ref, w_ref, degp_ref, hs_ref):
    h = jnp.dot(x_ref[...], w_ref[...], preferred_element_type=jnp.float32)
    deg = 1.0 + jnp.sum(degp_ref[...], axis=1)
    dinv = lax.rsqrt(deg)
    hs_ref[...] = h * dinv[:, None]


NB = 10
BN_ROWS = N_NODES // NB


def _hs_call(x, W, deg_parts):
    return pl.pallas_call(
        _hs_body,
        grid=(NB,),
        in_specs=[
            pl.BlockSpec((BN_ROWS, D), lambda i: (i, 0)),
            pl.BlockSpec((D, D), lambda i: (0, 0)),
            pl.BlockSpec((BN_ROWS, NC), lambda i: (i, 0)),
        ],
        out_specs=pl.BlockSpec((BN_ROWS, D), lambda i: (i, 0)),
        out_shape=jax.ShapeDtypeStruct((N_NODES, D), jnp.float32),
    )(x, W, deg_parts)


@functools.partial(
    pl.kernel,
    out_type=jax.ShapeDtypeStruct((NC, N_NODES, D), jnp.float32),
    mesh=_mesh,
    scratch_types=[
        pltpu.VMEM_SHARED((N_NODES, D), jnp.float32),
        [pltpu.VMEM((K,), jnp.int32) for _ in range(R)],
        [pltpu.VMEM((K,), jnp.int32) for _ in range(R)],
        [pltpu.VMEM((K, D), jnp.float32) for _ in range(R)],
        pltpu.VMEM((KT,), jnp.int32),
        pltpu.VMEM((KT,), jnp.int32),
        [pltpu.SemaphoreType.DMA for _ in range(R)],
        [pltpu.SemaphoreType.DMA for _ in range(R)],
        [pltpu.SemaphoreType.DMA for _ in range(R)],
    ],
)
def _acc_kernel(src_hbm, dst_hbm, hs_hbm, out_hbm,
                acc_sh, sidx, didx, rows, tsidx, tdidx, isem, gsem, ssem):
    cid = lax.axis_index("c")
    sid = lax.axis_index("s")
    wid = cid * NS + sid
    row0 = sid * ROWS_A
    base = wid * E_PER_TILE

    def _start_idx(o, b):
        c0 = base + (o * R + b) * K
        pltpu.async_copy(src_hbm.at[pl.ds(c0, K)], sidx[b], isem[b])
        pltpu.async_copy(dst_hbm.at[pl.ds(c0, K)], didx[b], isem[b])

    for b in range(R):
        _start_idx(0, b)

    zeros16 = jnp.zeros((16,), jnp.float32)
    for r in range(K):
        for c in range(D // 16):
            rows[0][r, pl.ds(c * 16, 16)] = zeros16

    nz_a = ROWS_A // K
    nz_b = ROWS_B // K

    @pl.when(sid < NS - 1)
    def _():
        for i in range(nz_a):
            pltpu.async_copy(rows[0], acc_sh.at[pl.ds(row0 + i * K, K)],
                             ssem[i % R])
        for i in range(nz_a):
            pltpu.make_async_copy(rows[0], acc_sh.at[pl.ds(row0, K)],
                                  ssem[i % R]).wait()

    @pl.when(sid == NS - 1)
    def _():
        for i in range(nz_b):
            pltpu.async_copy(rows[0], acc_sh.at[pl.ds(row0 + i * K, K)],
                             ssem[i % R])
        for i in range(nz_b):
            pltpu.make_async_copy(rows[0], acc_sh.at[pl.ds(row0, K)],
                                  ssem[i % R]).wait()

    plsc.subcore_barrier()

    def _outer(o, _):
        for b in range(R):
            pltpu.make_async_copy(src_hbm.at[pl.ds(base, K)], sidx[b],
                                  isem[b]).wait()
            pltpu.make_async_copy(dst_hbm.at[pl.ds(base, K)], didx[b],
                                  isem[b]).wait()
            pltpu.async_copy(hs_hbm.at[sidx[b]], rows[b], gsem[b])
        for b in range(R):
            pltpu.make_async_copy(hs_hbm.at[sidx[b]], rows[b], gsem[b]).wait()
            pltpu.async_copy(rows[b], acc_sh.at[didx[b]], ssem[b], add=True)
        for b in range(R):
            pltpu.make_async_copy(rows[b], acc_sh.at[didx[b]], ssem[b]).wait()

            @pl.when(o < NOUT - 1)
            def _():
                _start_idx(o + 1, b)

        return 0

    lax.fori_loop(0, NOUT, _outer, 0)

    t0 = base + NOUT * R * K
    pltpu.sync_copy(src_hbm.at[pl.ds(t0, KT)], tsidx)
    pltpu.sync_copy(dst_hbm.at[pl.ds(t0, KT)], tdidx)
    pltpu.async_copy(hs_hbm.at[tsidx], rows[0].at[pl.ds(0, KT)], gsem[0]).wait()
    pltpu.sync_copy(rows[0].at[pl.ds(0, KT)], acc_sh.at[tdidx], add=True)

    plsc.subcore_barrier()

    @pl.when(sid < NS - 1)
    def _():
        pltpu.sync_copy(acc_sh.at[pl.ds(row0, ROWS_A)],
                        out_hbm.at[cid, pl.ds(row0, ROWS_A)])

    @pl.when(sid == NS - 1)
    def _():
        pltpu.sync_copy(acc_sh.at[pl.ds(row0, ROWS_B)],
                        out_hbm.at[cid, pl.ds(row0, ROWS_B)])


def _final_body(accp_ref, x_ref, w_ref, degp_ref, batch_ref, b_ref, gamma_ref,
                beta_ref, mean_ref, var_ref, out_ref, sums_acc, counts_acc):
    deg = 1.0 + jnp.sum(degp_ref[...], axis=1)
    dinv = lax.rsqrt(deg)
    h = jnp.dot(x_ref[...], w_ref[...], preferred_element_type=jnp.float32)
    hs = h * dinv[:, None]
    acc = accp_ref[0] + accp_ref[1] + hs
    conv = acc * dinv[:, None] + b_ref[...]
    scale = gamma_ref[...] * lax.rsqrt(var_ref[...] + BN_EPS)
    shift = beta_ref[...] - mean_ref[...] * scale
    hbn = jnp.maximum(conv * scale + shift, 0.0)
    bt = batch_ref[...].reshape(BN_ROWS)
    onehot = (bt[None, :] == lax.broadcasted_iota(
        jnp.int32, (N_GRAPHS, BN_ROWS), 0)).astype(jnp.float32)
    sums = jnp.dot(onehot, hbn, preferred_element_type=jnp.float32)
    counts = jnp.sum(onehot, axis=1)

    i = pl.program_id(0)

    @pl.when(i == 0)
    def _():
        sums_acc[...] = jnp.zeros((N_GRAPHS, D), jnp.float32)
        counts_acc[...] = jnp.zeros((N_GRAPHS, D), jnp.float32)

    sums_acc[...] += sums
    counts_acc[...] += counts[:, None]

    @pl.when(i == NB - 1)
    def _():
        out_ref[...] = sums_acc[...] / jnp.clip(counts_acc[...], 1.0, None)


def _final_call(acc_parts, x, W, deg_parts, batch32, b, gamma, beta, mean, var):
    return pl.pallas_call(
        _final_body,
        grid=(NB,),
        in_specs=[
            pl.BlockSpec((NC, BN_ROWS, D), lambda i: (0, i, 0)),
            pl.BlockSpec((BN_ROWS, D), lambda i: (i, 0)),
            pl.BlockSpec((D, D), lambda i: (0, 0)),
            pl.BlockSpec((BN_ROWS, NC), lambda i: (i, 0)),
            pl.BlockSpec((1, 1, BN_ROWS), lambda i: (i, 0, 0)),
            pl.BlockSpec((1, D), lambda i: (0, 0)),
            pl.BlockSpec((1, D), lambda i: (0, 0)),
            pl.BlockSpec((1, D), lambda i: (0, 0)),
            pl.BlockSpec((1, D), lambda i: (0, 0)),
            pl.BlockSpec((1, D), lambda i: (0, 0)),
        ],
        out_specs=pl.BlockSpec((N_GRAPHS, D), lambda i: (0, 0)),
        out_shape=jax.ShapeDtypeStruct((N_GRAPHS, D), jnp.float32),
        scratch_shapes=[
            pltpu.VMEM((N_GRAPHS, D), jnp.float32),
            pltpu.VMEM((N_GRAPHS, D), jnp.float32),
        ],
    )(acc_parts, x, W, deg_parts, batch32.reshape(NB, 1, BN_ROWS),
      b.reshape(1, D), gamma.reshape(1, D), beta.reshape(1, D),
      mean.reshape(1, D), var.reshape(1, D))


def kernel(x, edge_index, batch, W, b, gamma, beta, running_mean, running_var):
    ei = edge_index.astype(jnp.int32)
    src = ei[0]
    dst = ei[1]
    batch32 = batch.astype(jnp.int32)

    deg_parts = _deg_kernel(dst).reshape(NC, N_NODES).T
    hs = _hs_call(x, W, deg_parts)
    acc_parts = _acc_kernel(src, dst, hs)
    return _final_call(acc_parts, x, W, deg_parts, batch32, b, gamma, beta,
                       running_mean, running_var)


# speedup vs baseline: 1.0339x; 1.0287x over previous
import functools

import jax
import jax.numpy as jnp
from jax import lax
from jax.experimental import pallas as pl
from jax.experimental.pallas import tpu as pltpu
from jax.experimental.pallas import tpu_sc as plsc

N_NODES = 10000
D = 128
N_GRAPHS = 64
N_EDGES = 320000
BN_EPS = 1e-5

NC = 2
NS = 16
NW = NC * NS
E_PER_TILE = N_EDGES // NW
K = 40
KT = 80
DK = 80
ROWS_A = 640
ROWS_B = 400

_mesh = plsc.VectorSubcoreMesh(core_axis_name="c", subcore_axis_name="s")


R = 8
DR = 5
NOUT = 31
DEG_NOUT = E_PER_TILE // (DR * DK)


@functools.partial(
    pl.kernel,
    out_type=jax.ShapeDtypeStruct((NC * N_NODES,), jnp.float32),
    mesh=_mesh,
    scratch_types=[
        pltpu.VMEM_SHARED((N_NODES,), jnp.float32),
        pltpu.VMEM((DK,), jnp.float32),
        [pltpu.VMEM((DK,), jnp.int32) for _ in range(DR)],
        [pltpu.SemaphoreType.DMA for _ in range(DR)],
        [pltpu.SemaphoreType.DMA for _ in range(DR)],
        pltpu.VMEM((N_NODES,), jnp.float32),
    ],
)
def _deg_kernel(dst_hbm, out_hbm, deg_sh, ones_v, didx, isem,
                ssem, stage_v):
    cid = lax.axis_index("c")
    sid = lax.axis_index("s")
    wid = cid * NS + sid

    ones16 = jnp.ones((16,), jnp.float32)
    for j in range(DK // 16):
        ones_v[pl.ds(j * 16, 16)] = ones16

    zeros16 = jnp.zeros((16,), jnp.float32)

    def _zb(i, _):
        stage_v[pl.ds(i * 16, 16)] = zeros16
        return 0

    lax.fori_loop(0, N_NODES // 16, _zb, 0)

    @pl.when(sid == 0)
    def _():
        pltpu.sync_copy(stage_v, deg_sh)

    base = wid * E_PER_TILE

    def _start_idx(o, b):
        c0 = base + (o * DR + b) * DK
        pltpu.async_copy(dst_hbm.at[pl.ds(c0, DK)], didx[b], isem[b])

    for b in range(DR):
        _start_idx(0, b)

    plsc.subcore_barrier()

    def _outer(o, _):
        for b in range(DR):
            pltpu.make_async_copy(dst_hbm.at[pl.ds(base, DK)], didx[b],
                                  isem[b]).wait()
            pltpu.async_copy(ones_v, deg_sh.at[didx[b]], ssem[b], add=True)
        for b in range(DR):
            pltpu.make_async_copy(ones_v, deg_sh.at[didx[b]], ssem[b]).wait()

            @pl.when(o < DEG_NOUT - 1)
            def _():
                _start_idx(o + 1, b)

        return 0

    lax.fori_loop(0, DEG_NOUT, _outer, 0)
    plsc.subcore_barrier()

    @pl.when(sid == 0)
    def _():
        pltpu.sync_copy(deg_sh, stage_v)
        pltpu.sync_copy(stage_v, out_hbm.at[pl.ds(cid * N_NODES, N_NODES)])


def _hs_body(x_ref, w_ref, degp_ref, hs_ref):
    h = jnp.dot(x_ref[...], w_ref[...], preferred_element_type=jnp.float32)
    deg = 1.0 + jnp.sum(degp_ref[...], axis=1)
    dinv = lax.rsqrt(deg)
    hs_ref[...] = h * dinv[:, None]


NB = 10
BN_ROWS = N_NODES // NB


def _hs_call(x, W, deg_parts):
    return pl.pallas_call(
        _hs_body,
        out_shape=jax.ShapeDtypeStruct((N_NODES, D), jnp.float32),
    )(x, W, deg_parts)


@functools.partial(
    pl.kernel,
    out_type=jax.ShapeDtypeStruct((NC, N_NODES, D), jnp.float32),
    mesh=_mesh,
    scratch_types=[
        pltpu.VMEM_SHARED((N_NODES, D), jnp.float32),
        [pltpu.VMEM((K,), jnp.int32) for _ in range(R)],
        [pltpu.VMEM((K,), jnp.int32) for _ in range(R)],
        [pltpu.VMEM((K, D), jnp.float32) for _ in range(R)],
        pltpu.VMEM((KT,), jnp.int32),
        pltpu.VMEM((KT,), jnp.int32),
        [pltpu.SemaphoreType.DMA for _ in range(R)],
        [pltpu.SemaphoreType.DMA for _ in range(R)],
        [pltpu.SemaphoreType.DMA for _ in range(R)],
    ],
)
def _acc_kernel(src_hbm, dst_hbm, hs_hbm, out_hbm,
                acc_sh, sidx, didx, rows, tsidx, tdidx, isem, gsem, ssem):
    cid = lax.axis_index("c")
    sid = lax.axis_index("s")
    wid = cid * NS + sid
    row0 = sid * ROWS_A
    base = wid * E_PER_TILE

    def _start_idx(o, b):
        c0 = base + (o * R + b) * K
        pltpu.async_copy(src_hbm.at[pl.ds(c0, K)], sidx[b], isem[b])
        pltpu.async_copy(dst_hbm.at[pl.ds(c0, K)], didx[b], isem[b])

    for b in range(R):
        _start_idx(0, b)

    zeros16 = jnp.zeros((16,), jnp.float32)
    for r in range(K):
        for c in range(D // 16):
            rows[0][r, pl.ds(c * 16, 16)] = zeros16

    nz_a = ROWS_A // K
    nz_b = ROWS_B // K

    @pl.when(sid < NS - 1)
    def _():
        for i in range(nz_a):
            pltpu.async_copy(rows[0], acc_sh.at[pl.ds(row0 + i * K, K)],
                             ssem[i % R])
        for i in range(nz_a):
            pltpu.make_async_copy(rows[0], acc_sh.at[pl.ds(row0, K)],
                                  ssem[i % R]).wait()

    @pl.when(sid == NS - 1)
    def _():
        for i in range(nz_b):
            pltpu.async_copy(rows[0], acc_sh.at[pl.ds(row0 + i * K, K)],
                             ssem[i % R])
        for i in range(nz_b):
            pltpu.make_async_copy(rows[0], acc_sh.at[pl.ds(row0, K)],
                                  ssem[i % R]).wait()

    plsc.subcore_barrier()

    def _outer(o, _):
        for b in range(R):
            pltpu.make_async_copy(src_hbm.at[pl.ds(base, K)], sidx[b],
                                  isem[b]).wait()
            pltpu.make_async_copy(dst_hbm.at[pl.ds(base, K)], didx[b],
                                  isem[b]).wait()
            pltpu.async_copy(hs_hbm.at[sidx[b]], rows[b], gsem[b])
        for b in range(R):
            pltpu.make_async_copy(hs_hbm.at[sidx[b]], rows[b], gsem[b]).wait()
            pltpu.async_copy(rows[b], acc_sh.at[didx[b]], ssem[b], add=True)
        for b in range(R):
            pltpu.make_async_copy(rows[b], acc_sh.at[didx[b]], ssem[b]).wait()

            @pl.when(o < NOUT - 1)
            def _():
                _start_idx(o + 1, b)

        return 0

    lax.fori_loop(0, NOUT, _outer, 0)

    t0 = base + NOUT * R * K
    pltpu.sync_copy(src_hbm.at[pl.ds(t0, KT)], tsidx)
    pltpu.sync_copy(dst_hbm.at[pl.ds(t0, KT)], tdidx)
    pltpu.async_copy(hs_hbm.at[tsidx], rows[0].at[pl.ds(0, KT)], gsem[0]).wait()
    pltpu.sync_copy(rows[0].at[pl.ds(0, KT)], acc_sh.at[tdidx], add=True)

    plsc.subcore_barrier()

    @pl.when(sid < NS - 1)
    def _():
        pltpu.sync_copy(acc_sh.at[pl.ds(row0, ROWS_A)],
                        out_hbm.at[cid, pl.ds(row0, ROWS_A)])

    @pl.when(sid == NS - 1)
    def _():
        pltpu.sync_copy(acc_sh.at[pl.ds(row0, ROWS_B)],
                        out_hbm.at[cid, pl.ds(row0, ROWS_B)])


def _final_body(accp_ref, x_ref, w_ref, degp_ref, batch_ref, b_ref, gamma_ref,
                beta_ref, mean_ref, var_ref, out_ref):
    deg = 1.0 + jnp.sum(degp_ref[...], axis=1)
    dinv = lax.rsqrt(deg)
    h = jnp.dot(x_ref[...], w_ref[...], preferred_element_type=jnp.float32)
    hs = h * dinv[:, None]
    acc = accp_ref[0] + accp_ref[1] + hs
    conv = acc * dinv[:, None] + b_ref[...]
    scale = gamma_ref[...] * lax.rsqrt(var_ref[...] + BN_EPS)
    shift = beta_ref[...] - mean_ref[...] * scale
    hbn = jnp.maximum(conv * scale + shift, 0.0)
    bt = batch_ref[...].reshape(N_NODES)
    onehot = (bt[None, :] == lax.broadcasted_iota(
        jnp.int32, (N_GRAPHS, N_NODES), 0)).astype(jnp.float32)
    sums = jnp.dot(onehot, hbn, preferred_element_type=jnp.float32)
    counts = jnp.clip(jnp.sum(onehot, axis=1), 1.0, None)
    out_ref[...] = sums / counts[:, None]


def _final_call(acc_parts, x, W, deg_parts, batch32, b, gamma, beta, mean, var):
    return pl.pallas_call(
        _final_body,
        out_shape=jax.ShapeDtypeStruct((N_GRAPHS, D), jnp.float32),
    )(acc_parts, x, W, deg_parts, batch32.reshape(1, N_NODES),
      b.reshape(1, D), gamma.reshape(1, D), beta.reshape(1, D),
      mean.reshape(1, D), var.reshape(1, D))


def kernel(x, edge_index, batch, W, b, gamma, beta, running_mean, running_var):
    ei = edge_index.astype(jnp.int32)
    src = ei[0]
    dst = ei[1]
    batch32 = batch.astype(jnp.int32)

    deg_parts = _deg_kernel(dst).reshape(NC, N_NODES).T
    hs = _hs_call(x, W, deg_parts)
    acc_parts = _acc_kernel(src, dst, hs)
    return _final_call(acc_parts, x, W, deg_parts, batch32, b, gamma, beta,
                       running_mean, running_var)


# speedup vs baseline: 1.0627x; 1.0278x over previous
import functools

import jax
import jax.numpy as jnp
from jax import lax
from jax.experimental import pallas as pl
from jax.experimental.pallas import tpu as pltpu
from jax.experimental.pallas import tpu_sc as plsc

N_NODES = 10000
D = 128
N_GRAPHS = 64
N_EDGES = 320000
BN_EPS = 1e-5

NC = 2
NS = 16
NW = NC * NS
E_PER_TILE = N_EDGES // NW
K = 40
KT = 80
DK = 80
ROWS_A = 640
ROWS_B = 400

_mesh = plsc.VectorSubcoreMesh(core_axis_name="c", subcore_axis_name="s")


R = 8
DR = 5
NOUT = 31
DEG_NOUT = E_PER_TILE // (DR * DK)


@functools.partial(
    pl.kernel,
    out_type=jax.ShapeDtypeStruct((NC * N_NODES,), jnp.float32),
    mesh=_mesh,
    scratch_types=[
        pltpu.VMEM_SHARED((N_NODES,), jnp.float32),
        pltpu.VMEM((DK,), jnp.float32),
        [pltpu.VMEM((DK,), jnp.int32) for _ in range(DR)],
        [pltpu.SemaphoreType.DMA for _ in range(DR)],
        [pltpu.SemaphoreType.DMA for _ in range(DR)],
        pltpu.VMEM((N_NODES,), jnp.float32),
    ],
)
def _deg_kernel(dst_hbm, out_hbm, deg_sh, ones_v, didx, isem,
                ssem, stage_v):
    cid = lax.axis_index("c")
    sid = lax.axis_index("s")
    wid = cid * NS + sid

    ones16 = jnp.ones((16,), jnp.float32)
    for j in range(DK // 16):
        ones_v[pl.ds(j * 16, 16)] = ones16

    zeros16 = jnp.zeros((16,), jnp.float32)

    def _zb(i, _):
        stage_v[pl.ds(i * 16, 16)] = zeros16
        return 0

    lax.fori_loop(0, N_NODES // 16, _zb, 0)

    @pl.when(sid == 0)
    def _():
        pltpu.sync_copy(stage_v, deg_sh)

    base = wid * E_PER_TILE

    def _start_idx(o, b):
        c0 = base + (o * DR + b) * DK
        pltpu.async_copy(dst_hbm.at[pl.ds(c0, DK)], didx[b], isem[b])

    for b in range(DR):
        _start_idx(0, b)

    plsc.subcore_barrier()

    def _outer(o, _):
        for b in range(DR):
            pltpu.make_async_copy(dst_hbm.at[pl.ds(base, DK)], didx[b],
                                  isem[b]).wait()
            pltpu.async_copy(ones_v, deg_sh.at[didx[b]], ssem[b], add=True)
        for b in range(DR):
            pltpu.make_async_copy(ones_v, deg_sh.at[didx[b]], ssem[b]).wait()

            @pl.when(o < DEG_NOUT - 1)
            def _():
                _start_idx(o + 1, b)

        return 0

    lax.fori_loop(0, DEG_NOUT, _outer, 0)
    plsc.subcore_barrier()

    @pl.when(sid == 0)
    def _():
        pltpu.sync_copy(deg_sh, stage_v)
        pltpu.sync_copy(stage_v, out_hbm.at[pl.ds(cid * N_NODES, N_NODES)])


def _hs_body(x_ref, w_ref, degp_ref, hs_ref):
    h = jnp.dot(x_ref[...], w_ref[...], preferred_element_type=jnp.float32)
    deg = 1.0 + jnp.sum(degp_ref[...], axis=0)
    dinv = lax.rsqrt(deg)
    hs_ref[...] = h * dinv[:, None]


NB = 10
BN_ROWS = N_NODES // NB


def _hs_call(x, W, deg_parts):
    return pl.pallas_call(
        _hs_body,
        out_shape=jax.ShapeDtypeStruct((N_NODES, D), jnp.float32),
    )(x, W, deg_parts)


@functools.partial(
    pl.kernel,
    out_type=jax.ShapeDtypeStruct((NC, N_NODES, D), jnp.float32),
    mesh=_mesh,
    scratch_types=[
        pltpu.VMEM_SHARED((N_NODES, D), jnp.float32),
        [pltpu.VMEM((K,), jnp.int32) for _ in range(R)],
        [pltpu.VMEM((K,), jnp.int32) for _ in range(R)],
        [pltpu.VMEM((K, D), jnp.float32) for _ in range(R)],
        pltpu.VMEM((KT,), jnp.int32),
        pltpu.VMEM((KT,), jnp.int32),
        [pltpu.SemaphoreType.DMA for _ in range(R)],
        [pltpu.SemaphoreType.DMA for _ in range(R)],
        [pltpu.SemaphoreType.DMA for _ in range(R)],
    ],
)
def _acc_kernel(src_hbm, dst_hbm, hs_hbm, out_hbm,
                acc_sh, sidx, didx, rows, tsidx, tdidx, isem, gsem, ssem):
    cid = lax.axis_index("c")
    sid = lax.axis_index("s")
    wid = cid * NS + sid
    row0 = sid * ROWS_A
    base = wid * E_PER_TILE

    def _start_idx(o, b):
        c0 = base + (o * R + b) * K
        pltpu.async_copy(src_hbm.at[pl.ds(c0, K)], sidx[b], isem[b])
        pltpu.async_copy(dst_hbm.at[pl.ds(c0, K)], didx[b], isem[b])

    for b in range(R):
        _start_idx(0, b)

    zeros16 = jnp.zeros((16,), jnp.float32)
    for r in range(K):
        for c in range(D // 16):
            rows[0][r, pl.ds(c * 16, 16)] = zeros16

    nz_a = ROWS_A // K
    nz_b = ROWS_B // K

    @pl.when(sid < NS - 1)
    def _():
        for i in range(nz_a):
            pltpu.async_copy(rows[0], acc_sh.at[pl.ds(row0 + i * K, K)],
                             ssem[i % R])
        for i in range(nz_a):
            pltpu.make_async_copy(rows[0], acc_sh.at[pl.ds(row0, K)],
                                  ssem[i % R]).wait()

    @pl.when(sid == NS - 1)
    def _():
        for i in range(nz_b):
            pltpu.async_copy(rows[0], acc_sh.at[pl.ds(row0 + i * K, K)],
                             ssem[i % R])
        for i in range(nz_b):
            pltpu.make_async_copy(rows[0], acc_sh.at[pl.ds(row0, K)],
                                  ssem[i % R]).wait()

    plsc.subcore_barrier()

    def _outer(o, _):
        for b in range(R):
            pltpu.make_async_copy(src_hbm.at[pl.ds(base, K)], sidx[b],
                                  isem[b]).wait()
            pltpu.make_async_copy(dst_hbm.at[pl.ds(base, K)], didx[b],
                                  isem[b]).wait()
            pltpu.async_copy(hs_hbm.at[sidx[b]], rows[b], gsem[b])
        for b in range(R):
            pltpu.make_async_copy(hs_hbm.at[sidx[b]], rows[b], gsem[b]).wait()
            pltpu.async_copy(rows[b], acc_sh.at[didx[b]], ssem[b], add=True)
        for b in range(R):
            pltpu.make_async_copy(rows[b], acc_sh.at[didx[b]], ssem[b]).wait()

            @pl.when(o < NOUT - 1)
            def _():
                _start_idx(o + 1, b)

        return 0

    lax.fori_loop(0, NOUT, _outer, 0)

    t0 = base + NOUT * R * K
    pltpu.sync_copy(src_hbm.at[pl.ds(t0, KT)], tsidx)
    pltpu.sync_copy(dst_hbm.at[pl.ds(t0, KT)], tdidx)
    pltpu.async_copy(hs_hbm.at[tsidx], rows[0].at[pl.ds(0, KT)], gsem[0]).wait()
    pltpu.sync_copy(rows[0].at[pl.ds(0, KT)], acc_sh.at[tdidx], add=True)

    plsc.subcore_barrier()

    @pl.when(sid < NS - 1)
    def _():
        pltpu.sync_copy(acc_sh.at[pl.ds(row0, ROWS_A)],
                        out_hbm.at[cid, pl.ds(row0, ROWS_A)])

    @pl.when(sid == NS - 1)
    def _():
        pltpu.sync_copy(acc_sh.at[pl.ds(row0, ROWS_B)],
                        out_hbm.at[cid, pl.ds(row0, ROWS_B)])


def _final_body(accp_ref, x_ref, w_ref, degp_ref, batch_ref, b_ref, gamma_ref,
                beta_ref, mean_ref, var_ref, out_ref):
    deg = 1.0 + jnp.sum(degp_ref[...], axis=0)
    dinv = lax.rsqrt(deg)
    h = jnp.dot(x_ref[...], w_ref[...], preferred_element_type=jnp.float32)
    hs = h * dinv[:, None]
    acc = accp_ref[0] + accp_ref[1] + hs
    conv = acc * dinv[:, None] + b_ref[...]
    scale = gamma_ref[...] * lax.rsqrt(var_ref[...] + BN_EPS)
    shift = beta_ref[...] - mean_ref[...] * scale
    hbn = jnp.maximum(conv * scale + shift, 0.0)
    bt = batch_ref[...].reshape(N_NODES)
    onehot = (bt[None, :] == lax.broadcasted_iota(
        jnp.int32, (N_GRAPHS, N_NODES), 0)).astype(jnp.float32)
    sums = jnp.dot(onehot, hbn, preferred_element_type=jnp.float32)
    counts = jnp.clip(jnp.sum(onehot, axis=1), 1.0, None)
    out_ref[...] = sums / counts[:, None]


def _final_call(acc_parts, x, W, deg_parts, batch32, b, gamma, beta, mean, var):
    return pl.pallas_call(
        _final_body,
        out_shape=jax.ShapeDtypeStruct((N_GRAPHS, D), jnp.float32),
    )(acc_parts, x, W, deg_parts, batch32.reshape(1, N_NODES),
      b.reshape(1, D), gamma.reshape(1, D), beta.reshape(1, D),
      mean.reshape(1, D), var.reshape(1, D))


def kernel(x, edge_index, batch, W, b, gamma, beta, running_mean, running_var):
    ei = edge_index.astype(jnp.int32)
    src = ei[0]
    dst = ei[1]
    batch32 = batch.astype(jnp.int32)

    deg_parts = _deg_kernel(dst).reshape(NC, N_NODES)
    hs = _hs_call(x, W, deg_parts)
    acc_parts = _acc_kernel(src, dst, hs)
    return _final_call(acc_parts, x, W, deg_parts, batch32, b, gamma, beta,
                       running_mean, running_var)
